# Initial kernel scaffold; baseline (speedup 1.0000x reference)
import jax
import jax.numpy as jnp
from jax.experimental import pallas as pl


def kernel(x, edge_index, W1, b1, W2, b2):
    raise NotImplementedError("write your pallas kernel here")


# speedup vs baseline: 19.9129x; 19.9129x over previous
import functools

import jax
import jax.numpy as jnp
from jax import lax
from jax.experimental import pallas as pl
from jax.experimental.pallas import tpu as pltpu
from jax.experimental.pallas import tpu_sc as plsc

N = 10000
E = 320000
D = 128

NC = 2
NS = 16
NW = NC * NS
EPW = E // NW
CW = 125
NCH = EPW // CW
NP = 10240
RPT = NP // NS


@functools.cache
def _sc_kernels():
    mesh = plsc.VectorSubcoreMesh(core_axis_name="c", subcore_axis_name="s",
                                  num_cores=NC, num_subcores=NS)

    @functools.partial(
        pl.kernel,
        out_type=jax.ShapeDtypeStruct((NC, NP, D), jnp.float32),
        mesh=mesh,
        scratch_types=[
            pltpu.VMEM((NCH, CW), jnp.int32),
            pltpu.VMEM((CW, D), jnp.float32),
            pltpu.VMEM_SHARED((NP, D), jnp.float32),
            pltpu.SemaphoreType.DMA,
        ],
    )
    def deg_kernel(dst_hbm, ones_hbm, zeros_hbm, out_hbm,
                   dst_v, ones_v, acc, sem):
        c = lax.axis_index("c")
        s = lax.axis_index("s")
        wid = c * NS + s
        pltpu.sync_copy(zeros_hbm, acc.at[pl.ds(s * RPT, RPT)])
        pltpu.sync_copy(ones_hbm, ones_v)
        pltpu.sync_copy(dst_hbm.at[wid], dst_v)
        plsc.subcore_barrier()

        def body(j, carry):
            pltpu.sync_copy(ones_v, acc.at[dst_v.at[j]], add=True)
            return carry

        lax.fori_loop(0, NCH, body, 0)
        plsc.subcore_barrier()
        pltpu.sync_copy(acc.at[pl.ds(s * RPT, RPT)],
                        out_hbm.at[c, pl.ds(s * RPT, RPT)])

    @functools.partial(
        pl.kernel,
        out_type=jax.ShapeDtypeStruct((NC, NP, D), jnp.float32),
        mesh=mesh,
        scratch_types=[
            pltpu.VMEM((NCH, CW), jnp.int32),
            pltpu.VMEM((NCH, CW), jnp.int32),
            pltpu.VMEM((CW, D), jnp.float32),
            pltpu.VMEM_SHARED((NP, D), jnp.float32),
            pltpu.SemaphoreType.DMA,
        ],
    )
    def agg_kernel(ht_hbm, src_hbm, dst_hbm, zeros_hbm, out_hbm,
                   src_v, dst_v, rows_v, acc, sem):
        c = lax.axis_index("c")
        s = lax.axis_index("s")
        wid = c * NS + s
        pltpu.sync_copy(zeros_hbm, acc.at[pl.ds(s * RPT, RPT)])
        pltpu.sync_copy(src_hbm.at[wid], src_v)
        pltpu.sync_copy(dst_hbm.at[wid], dst_v)
        plsc.subcore_barrier()

        def body(j, carry):
            pltpu.async_copy(ht_hbm.at[src_v.at[j]], rows_v, sem).wait()
            pltpu.sync_copy(rows_v, acc.at[dst_v.at[j]], add=True)
            return carry

        lax.fori_loop(0, NCH, body, 0)
        plsc.subcore_barrier()
        pltpu.sync_copy(acc.at[pl.ds(s * RPT, RPT)],
                        out_hbm.at[c, pl.ds(s * RPT, RPT)])

    return deg_kernel, agg_kernel


_BLK = 1000
_GRID = N // _BLK


def _dinv_of(degp):
    deg = degp[0, :, 0:1] + degp[1, :, 0:1] + 1.0
    return lax.rsqrt(deg)


def _pre_body(x_ref, w_ref, degp_ref, ht_ref):
    dinv = _dinv_of(degp_ref)
    ht_ref[...] = jnp.dot(x_ref[...], w_ref[...],
                          preferred_element_type=jnp.float32) * dinv


def _mid_body(aggp_ref, ht1_ref, degp_ref, w2_ref, b1_ref, h_ref, ht2_ref):
    dinv = _dinv_of(degp_ref)
    x1 = dinv * (aggp_ref[0] + aggp_ref[1] + ht1_ref[...]) + b1_ref[...]
    h = jnp.maximum(x1, 0.0)
    h_ref[...] = h
    ht2_ref[...] = jnp.dot(h, w2_ref[...],
                           preferred_element_type=jnp.float32) * dinv


def _post_body(aggp_ref, ht2_ref, degp_ref, b2_ref, out_ref):
    dinv = _dinv_of(degp_ref)
    out_ref[...] = dinv * (aggp_ref[0] + aggp_ref[1] + ht2_ref[...]) + b2_ref[...]


_spec_rows = pl.BlockSpec((_BLK, D), lambda i: (i, 0))
_spec_degp = pl.BlockSpec((NC, _BLK, D), lambda i: (0, i, 0))
_spec_aggp = pl.BlockSpec((NC, _BLK, D), lambda i: (0, i, 0))
_spec_w = pl.BlockSpec((D, D), lambda i: (0, 0))
_spec_b = pl.BlockSpec((1, D), lambda i: (0, 0))

_pre = pl.pallas_call(
    _pre_body,
    grid=(_GRID,),
    in_specs=[_spec_rows, _spec_w, _spec_degp],
    out_specs=_spec_rows,
    out_shape=jax.ShapeDtypeStruct((N, D), jnp.float32),
)

_mid = pl.pallas_call(
    _mid_body,
    grid=(_GRID,),
    in_specs=[_spec_aggp, _spec_rows, _spec_degp, _spec_w, _spec_b],
    out_specs=[_spec_rows, _spec_rows],
    out_shape=[jax.ShapeDtypeStruct((N, D), jnp.float32),
               jax.ShapeDtypeStruct((N, D), jnp.float32)],
)

_post = pl.pallas_call(
    _post_body,
    grid=(_GRID,),
    in_specs=[_spec_aggp, _spec_rows, _spec_degp, _spec_b],
    out_specs=_spec_rows,
    out_shape=jax.ShapeDtypeStruct((N, D), jnp.float32),
)


def kernel(x, edge_index, W1, b1, W2, b2):
    src = edge_index[0].reshape(NW, NCH, CW)
    dst = edge_index[1].reshape(NW, NCH, CW)
    onesrows = jnp.ones((CW, D), jnp.float32)
    zrows = jnp.zeros((RPT, D), jnp.float32)
    b1r = b1.reshape(1, D)
    b2r = b2.reshape(1, D)

    deg_kernel, agg_kernel = _sc_kernels()
    degp = deg_kernel(dst, onesrows, zrows)
    ht1 = _pre(x, W1, degp)
    aggp1 = agg_kernel(ht1, src, dst, zrows)
    h, ht2 = _mid(aggp1, ht1, degp, W2, b1r)
    aggp2 = agg_kernel(ht2, src, dst, zrows)
    out = _post(aggp2, ht2, degp, b2r)
    return (out, h)


# speedup vs baseline: 22.1233x; 1.1110x over previous
import functools

import jax
import jax.numpy as jnp
from jax import lax
from jax.experimental import pallas as pl
from jax.experimental.pallas import tpu as pltpu
from jax.experimental.pallas import tpu_sc as plsc

N = 10000
E = 320000
D = 128

NC = 2
NS = 16
NW = NC * NS
EPW = E // NW
CW = 125
NCH = EPW // CW
HALF = NCH // 2
NP = 10240
RPT = NP // NS


@functools.cache
def _sc_kernels():
    mesh = plsc.VectorSubcoreMesh(core_axis_name="c", subcore_axis_name="s",
                                  num_cores=NC, num_subcores=NS)

    @functools.partial(
        pl.kernel,
        out_type=jax.ShapeDtypeStruct((NC, NP, D), jnp.float32),
        mesh=mesh,
        scratch_types=[
            pltpu.VMEM((NCH, CW), jnp.int32),
            pltpu.VMEM((CW, D), jnp.float32),
            pltpu.VMEM_SHARED((NP, D), jnp.float32),
            pltpu.SemaphoreType.DMA,
        ],
    )
    def deg_kernel(dst_hbm, ones_hbm, zeros_hbm, out_hbm,
                   dst_v, ones_v, acc, sem):
        c = lax.axis_index("c")
        s = lax.axis_index("s")
        wid = c * NS + s
        pltpu.sync_copy(zeros_hbm, acc.at[pl.ds(s * RPT, RPT)])
        pltpu.sync_copy(ones_hbm, ones_v)
        pltpu.sync_copy(dst_hbm.at[wid], dst_v)
        plsc.subcore_barrier()

        def body(j, carry):
            pltpu.sync_copy(ones_v, acc.at[dst_v.at[j]], add=True)
            return carry

        lax.fori_loop(0, NCH, body, 0)
        plsc.subcore_barrier()
        pltpu.sync_copy(acc.at[pl.ds(s * RPT, RPT)],
                        out_hbm.at[c, pl.ds(s * RPT, RPT)])

    @functools.partial(
        pl.kernel,
        out_type=jax.ShapeDtypeStruct((NC, NP, D), jnp.float32),
        mesh=mesh,
        scratch_types=[
            pltpu.VMEM((HALF, CW), jnp.int32),
            pltpu.VMEM((HALF, CW), jnp.int32),
            pltpu.VMEM((CW, D), jnp.float32),
            pltpu.VMEM((CW, D), jnp.float32),
            pltpu.VMEM_SHARED((NP, D), jnp.float32),
            pltpu.SemaphoreType.DMA,
            pltpu.SemaphoreType.DMA,
        ],
    )
    def agg_kernel(ht_hbm, src_hbm, dst_hbm, zeros_hbm, out_hbm,
                   src_v, dst_v, b0, b1, acc, s0, s1):
        c = lax.axis_index("c")
        s = lax.axis_index("s")
        wid = c * NS + s
        pltpu.sync_copy(zeros_hbm, acc.at[pl.ds(s * RPT, RPT)])
        plsc.subcore_barrier()

        def body(jj, carry):
            j0 = 2 * jj
            j1 = j0 + 1
            c0 = pltpu.async_copy(ht_hbm.at[src_v.at[j0]], b0, s0)
            c1 = pltpu.async_copy(ht_hbm.at[src_v.at[j1]], b1, s1)
            c0.wait()
            pltpu.sync_copy(b0, acc.at[dst_v.at[j0]], add=True)
            c1.wait()
            pltpu.sync_copy(b1, acc.at[dst_v.at[j1]], add=True)
            return carry

        for h in range(2):
            pltpu.sync_copy(src_hbm.at[wid, pl.ds(h * HALF, HALF)], src_v)
            pltpu.sync_copy(dst_hbm.at[wid, pl.ds(h * HALF, HALF)], dst_v)
            lax.fori_loop(0, HALF // 2, body, 0)
        plsc.subcore_barrier()
        pltpu.sync_copy(acc.at[pl.ds(s * RPT, RPT)],
                        out_hbm.at[c, pl.ds(s * RPT, RPT)])

    return deg_kernel, agg_kernel


_BLK = 1000
_GRID = N // _BLK


def _dinv_of(degp):
    deg = degp[0, :, 0:1] + degp[1, :, 0:1] + 1.0
    return lax.rsqrt(deg)


def _pre_body(x_ref, w_ref, degp_ref, ht_ref):
    dinv = _dinv_of(degp_ref)
    ht_ref[...] = jnp.dot(x_ref[...], w_ref[...],
                          preferred_element_type=jnp.float32) * dinv


def _mid_body(aggp_ref, ht1_ref, degp_ref, w2_ref, b1_ref, h_ref, ht2_ref):
    dinv = _dinv_of(degp_ref)
    x1 = dinv * (aggp_ref[0] + aggp_ref[1] + ht1_ref[...]) + b1_ref[...]
    h = jnp.maximum(x1, 0.0)
    h_ref[...] = h
    ht2_ref[...] = jnp.dot(h, w2_ref[...],
                           preferred_element_type=jnp.float32) * dinv


def _post_body(aggp_ref, ht2_ref, degp_ref, b2_ref, out_ref):
    dinv = _dinv_of(degp_ref)
    out_ref[...] = dinv * (aggp_ref[0] + aggp_ref[1] + ht2_ref[...]) + b2_ref[...]


_spec_rows = pl.BlockSpec((_BLK, D), lambda i: (i, 0))
_spec_degp = pl.BlockSpec((NC, _BLK, D), lambda i: (0, i, 0))
_spec_aggp = pl.BlockSpec((NC, _BLK, D), lambda i: (0, i, 0))
_spec_w = pl.BlockSpec((D, D), lambda i: (0, 0))
_spec_b = pl.BlockSpec((1, D), lambda i: (0, 0))

_pre = pl.pallas_call(
    _pre_body,
    grid=(_GRID,),
    in_specs=[_spec_rows, _spec_w, _spec_degp],
    out_specs=_spec_rows,
    out_shape=jax.ShapeDtypeStruct((N, D), jnp.float32),
)

_mid = pl.pallas_call(
    _mid_body,
    grid=(_GRID,),
    in_specs=[_spec_aggp, _spec_rows, _spec_degp, _spec_w, _spec_b],
    out_specs=[_spec_rows, _spec_rows],
    out_shape=[jax.ShapeDtypeStruct((N, D), jnp.float32),
               jax.ShapeDtypeStruct((N, D), jnp.float32)],
)

_post = pl.pallas_call(
    _post_body,
    grid=(_GRID,),
    in_specs=[_spec_aggp, _spec_rows, _spec_degp, _spec_b],
    out_specs=_spec_rows,
    out_shape=jax.ShapeDtypeStruct((N, D), jnp.float32),
)


def kernel(x, edge_index, W1, b1, W2, b2):
    src = edge_index[0].reshape(NW, NCH, CW)
    dst = edge_index[1].reshape(NW, NCH, CW)
    onesrows = jnp.ones((CW, D), jnp.float32)
    zrows = jnp.zeros((RPT, D), jnp.float32)
    b1r = b1.reshape(1, D)
    b2r = b2.reshape(1, D)

    deg_kernel, agg_kernel = _sc_kernels()
    degp = deg_kernel(dst, onesrows, zrows)
    ht1 = _pre(x, W1, degp)
    aggp1 = agg_kernel(ht1, src, dst, zrows)
    h, ht2 = _mid(aggp1, ht1, degp, W2, b1r)
    aggp2 = agg_kernel(ht2, src, dst, zrows)
    out = _post(aggp2, ht2, degp, b2r)
    return (out, h)


# speedup vs baseline: 26.8628x; 1.2142x over previous
import functools

import jax
import jax.numpy as jnp
from jax import lax
from jax.experimental import pallas as pl
from jax.experimental.pallas import tpu as pltpu
from jax.experimental.pallas import tpu_sc as plsc

N = 10000
E = 320000
D = 128

NC = 2
NS = 16
NW = NC * NS
EPW = E // NW
CW = 125
NCH = EPW // CW
HALF = NCH // 2
NP = 10240
RPT = NP // NS


@functools.cache
def _sc_kernels():
    mesh = plsc.VectorSubcoreMesh(core_axis_name="c", subcore_axis_name="s",
                                  num_cores=NC, num_subcores=NS)

    @functools.partial(
        pl.kernel,
        out_type=jax.ShapeDtypeStruct((NC, NP, D), jnp.float32),
        mesh=mesh,
        scratch_types=[
            pltpu.VMEM((NCH, CW), jnp.int32),
            pltpu.VMEM((CW, D), jnp.float32),
            pltpu.VMEM_SHARED((NP, D), jnp.float32),
            pltpu.SemaphoreType.DMA,
        ],
    )
    def deg_kernel(dst_hbm, ones_hbm, zeros_hbm, out_hbm,
                   dst_v, ones_v, acc, sem):
        c = lax.axis_index("c")
        s = lax.axis_index("s")
        wid = c * NS + s
        pltpu.sync_copy(zeros_hbm, acc.at[pl.ds(s * RPT, RPT)])
        pltpu.sync_copy(ones_hbm, ones_v)
        pltpu.sync_copy(dst_hbm.at[wid], dst_v)
        plsc.subcore_barrier()

        def body(j, carry):
            pltpu.sync_copy(ones_v, acc.at[dst_v.at[j]], add=True)
            return carry

        lax.fori_loop(0, NCH, body, 0)
        plsc.subcore_barrier()
        pltpu.sync_copy(acc.at[pl.ds(s * RPT, RPT)],
                        out_hbm.at[c, pl.ds(s * RPT, RPT)])

    @functools.partial(
        pl.kernel,
        out_type=jax.ShapeDtypeStruct((NC, NP, D), jnp.float32),
        mesh=mesh,
        scratch_types=[
            pltpu.VMEM((HALF, CW), jnp.int32),
            pltpu.VMEM((HALF, CW), jnp.int32),
            pltpu.VMEM((CW, D), jnp.float32),
            pltpu.VMEM((CW, D), jnp.float32),
            pltpu.VMEM_SHARED((NP, D), jnp.float32),
            pltpu.SemaphoreType.DMA,
            pltpu.SemaphoreType.DMA,
        ],
    )
    def agg_kernel(ht_hbm, src_hbm, dst_hbm, zeros_hbm, out_hbm,
                   src_v, dst_v, b0, b1, acc, s0, s1):
        c = lax.axis_index("c")
        s = lax.axis_index("s")
        wid = c * NS + s
        pltpu.sync_copy(zeros_hbm, acc.at[pl.ds(s * RPT, RPT)])
        plsc.subcore_barrier()

        def body(jj, carry):
            j0 = 2 * jj
            j1 = j0 + 1
            j2 = jnp.minimum(j0 + 2, HALF - 1)
            pltpu.async_copy(ht_hbm.at[src_v.at[j1]], b1, s1)
            pltpu.make_async_copy(ht_hbm.at[src_v.at[j0]], b0, s0).wait()
            pltpu.sync_copy(b0, acc.at[dst_v.at[j0]], add=True)
            pltpu.async_copy(ht_hbm.at[src_v.at[j2]], b0, s0)
            pltpu.make_async_copy(ht_hbm.at[src_v.at[j1]], b1, s1).wait()
            pltpu.sync_copy(b1, acc.at[dst_v.at[j1]], add=True)
            return carry

        for h in range(2):
            pltpu.sync_copy(src_hbm.at[wid, pl.ds(h * HALF, HALF)], src_v)
            pltpu.sync_copy(dst_hbm.at[wid, pl.ds(h * HALF, HALF)], dst_v)
            pltpu.async_copy(ht_hbm.at[src_v.at[0]], b0, s0)
            lax.fori_loop(0, HALF // 2, body, 0)
            pltpu.make_async_copy(ht_hbm.at[src_v.at[HALF - 1]], b0, s0).wait()
        plsc.subcore_barrier()
        pltpu.sync_copy(acc.at[pl.ds(s * RPT, RPT)],
                        out_hbm.at[c, pl.ds(s * RPT, RPT)])

    return deg_kernel, agg_kernel


_BLK = 1000
_GRID = N // _BLK


def _dinv_of(degp):
    deg = degp[0, :, 0:1] + degp[1, :, 0:1] + 1.0
    return lax.rsqrt(deg)


def _pre_body(x_ref, w_ref, degp_ref, ht_ref):
    dinv = _dinv_of(degp_ref)
    ht_ref[...] = jnp.dot(x_ref[...], w_ref[...],
                          preferred_element_type=jnp.float32) * dinv


def _mid_body(aggp_ref, ht1_ref, degp_ref, w2_ref, b1_ref, h_ref, ht2_ref):
    dinv = _dinv_of(degp_ref)
    x1 = dinv * (aggp_ref[0] + aggp_ref[1] + ht1_ref[...]) + b1_ref[...]
    h = jnp.maximum(x1, 0.0)
    h_ref[...] = h
    ht2_ref[...] = jnp.dot(h, w2_ref[...],
                           preferred_element_type=jnp.float32) * dinv


def _post_body(aggp_ref, ht2_ref, degp_ref, b2_ref, out_ref):
    dinv = _dinv_of(degp_ref)
    out_ref[...] = dinv * (aggp_ref[0] + aggp_ref[1] + ht2_ref[...]) + b2_ref[...]


_spec_rows = pl.BlockSpec((_BLK, D), lambda i: (i, 0))
_spec_degp = pl.BlockSpec((NC, _BLK, D), lambda i: (0, i, 0))
_spec_aggp = pl.BlockSpec((NC, _BLK, D), lambda i: (0, i, 0))
_spec_w = pl.BlockSpec((D, D), lambda i: (0, 0))
_spec_b = pl.BlockSpec((1, D), lambda i: (0, 0))

_pre = pl.pallas_call(
    _pre_body,
    grid=(_GRID,),
    in_specs=[_spec_rows, _spec_w, _spec_degp],
    out_specs=_spec_rows,
    out_shape=jax.ShapeDtypeStruct((N, D), jnp.float32),
)

_mid = pl.pallas_call(
    _mid_body,
    grid=(_GRID,),
    in_specs=[_spec_aggp, _spec_rows, _spec_degp, _spec_w, _spec_b],
    out_specs=[_spec_rows, _spec_rows],
    out_shape=[jax.ShapeDtypeStruct((N, D), jnp.float32),
               jax.ShapeDtypeStruct((N, D), jnp.float32)],
)

_post = pl.pallas_call(
    _post_body,
    grid=(_GRID,),
    in_specs=[_spec_aggp, _spec_rows, _spec_degp, _spec_b],
    out_specs=_spec_rows,
    out_shape=jax.ShapeDtypeStruct((N, D), jnp.float32),
)


def kernel(x, edge_index, W1, b1, W2, b2):
    src = edge_index[0].reshape(NW, NCH, CW)
    dst = edge_index[1].reshape(NW, NCH, CW)
    onesrows = jnp.ones((CW, D), jnp.float32)
    zrows = jnp.zeros((RPT, D), jnp.float32)
    b1r = b1.reshape(1, D)
    b2r = b2.reshape(1, D)

    deg_kernel, agg_kernel = _sc_kernels()
    degp = deg_kernel(dst, onesrows, zrows)
    ht1 = _pre(x, W1, degp)
    aggp1 = agg_kernel(ht1, src, dst, zrows)
    h, ht2 = _mid(aggp1, ht1, degp, W2, b1r)
    aggp2 = agg_kernel(ht2, src, dst, zrows)
    out = _post(aggp2, ht2, degp, b2r)
    return (out, h)
